# baseline (speedup 1.0000x reference)
import functools

import jax
import jax.numpy as jnp
from jax.experimental import pallas as pl
from jax.experimental.pallas import tpu as pltpu

_N = 4096
_TP = 256
_KA = 8


def _chamfer_body(p_ref, g_ref, o1_ref, o2_ref):
    ga = g_ref[0]

    def step(c, carry):
        cacc, s1 = carry
        pc = p_ref[0, pl.ds(c * _TP, _TP), :]
        d = jax.lax.dot_general(
            pc, ga, (((1,), (0,)), ((), ())),
            preferred_element_type=jnp.float32)
        s1 = s1 + jnp.sum(jnp.min(d, axis=1))
        cacc = jnp.minimum(cacc, jnp.min(d, axis=0, keepdims=True))
        return cacc, s1

    cacc0 = jnp.full((1, _N), jnp.inf, dtype=jnp.float32)
    cacc, s1 = jax.lax.fori_loop(0, _N // _TP, step, (cacc0, jnp.float32(0.0)))
    s2 = jnp.sum(cacc)
    o1_ref[0, 0, :] = jnp.full((128,), s1, dtype=jnp.float32)
    o2_ref[0, 0, :] = jnp.full((128,), s2, dtype=jnp.float32)


def _voxelize(coord):
    coord_no_nan = jnp.where(jnp.isnan(coord), jnp.inf, coord)
    global_min = jnp.min(coord_no_nan, axis=1, keepdims=True)
    grid_coord = (coord - global_min) / 0.1
    return grid_coord.astype(jnp.int32).astype(jnp.float32)


@jax.jit
def kernel(preds, gts):
    pv = _voxelize(preds)
    gv = _voxelize(gts)
    shift = jnp.floor(jnp.maximum(jnp.max(pv, axis=1, keepdims=True),
                                  jnp.max(gv, axis=1, keepdims=True)) * 0.5)
    pv = pv - shift
    gv = gv - shift
    p8 = jnp.concatenate([preds, pv], axis=0)
    g8 = jnp.concatenate([gts, gv], axis=0)

    def _split(sq):
        hi = jnp.floor(sq * (1.0 / 256.0)) * 256.0
        return hi, sq - hi
    xx = jnp.sum(p8 * p8, axis=2, keepdims=True)
    yy = jnp.sum(g8 * g8, axis=2, keepdims=True)
    xxh, xxl = _split(xx)
    yyh, yyl = _split(yy)
    ones = jnp.ones((8, _N, 1), jnp.float32)
    zero = jnp.zeros((8, _N, 1), jnp.float32)
    pa8 = jnp.concatenate(
        [p8, xxh, xxl, ones, ones, zero], axis=2)
    ga8 = jnp.concatenate(
        [-2.0 * g8, ones, ones, yyh, yyl, zero], axis=2)
    ga8 = ga8.transpose(0, 2, 1)

    s1, s2 = pl.pallas_call(
        _chamfer_body,
        grid=(8,),
        in_specs=[
            pl.BlockSpec((1, _N, _KA), lambda b: (b, 0, 0)),
            pl.BlockSpec((1, _KA, _N), lambda b: (b, 0, 0)),
        ],
        out_specs=[
            pl.BlockSpec((1, 1, 128), lambda b: (b, 0, 0)),
            pl.BlockSpec((1, 1, 128), lambda b: (b, 0, 0)),
        ],
        out_shape=[
            jax.ShapeDtypeStruct((8, 1, 128), jnp.float32),
            jax.ShapeDtypeStruct((8, 1, 128), jnp.float32),
        ],
    )(pa8, ga8)

    total = jnp.sum(s1[:, 0, 0]) + jnp.sum(s2[:, 0, 0])
    return total / jnp.float32(4 * _N)


# speedup vs baseline: 1.4093x; 1.4093x over previous
import functools

import jax
import jax.numpy as jnp
from jax.experimental import pallas as pl
from jax.experimental.pallas import tpu as pltpu

_N = 4096
_TP = 256
_KA = 8


def _chamfer_body(p_ref, g_ref, o1_ref, o2_ref):
    ga = g_ref[0]

    def step(c, carry):
        cacc, s1 = carry
        pc = p_ref[0, pl.ds(c * _TP, _TP), :]
        d = jax.lax.dot_general(
            pc, ga, (((1,), (0,)), ((), ())),
            preferred_element_type=jnp.float32)
        s1 = s1 + jnp.sum(jnp.min(d, axis=1))
        cacc = jnp.minimum(cacc, jnp.min(d, axis=0, keepdims=True))
        return cacc, s1

    cacc0 = jnp.full((1, _N), jnp.inf, dtype=jnp.float32)
    cacc, s1 = jax.lax.fori_loop(0, _N // _TP, step, (cacc0, jnp.float32(0.0)),
                                 unroll=16)
    s2 = jnp.sum(cacc)
    o1_ref[0, 0, :] = jnp.full((128,), s1, dtype=jnp.float32)
    o2_ref[0, 0, :] = jnp.full((128,), s2, dtype=jnp.float32)


def _voxelize(coord):
    coord_no_nan = jnp.where(jnp.isnan(coord), jnp.inf, coord)
    global_min = jnp.min(coord_no_nan, axis=1, keepdims=True)
    grid_coord = (coord - global_min) / 0.1
    return grid_coord.astype(jnp.int32).astype(jnp.float32)


@jax.jit
def kernel(preds, gts):
    pv = _voxelize(preds)
    gv = _voxelize(gts)
    shift = jnp.floor(jnp.maximum(jnp.max(pv, axis=1, keepdims=True),
                                  jnp.max(gv, axis=1, keepdims=True)) * 0.5)
    pv = pv - shift
    gv = gv - shift
    p8 = jnp.concatenate([preds, pv], axis=0)
    g8 = jnp.concatenate([gts, gv], axis=0)

    def _split(sq):
        hi = jnp.floor(sq * (1.0 / 256.0)) * 256.0
        return hi, sq - hi
    xx = jnp.sum(p8 * p8, axis=2, keepdims=True)
    yy = jnp.sum(g8 * g8, axis=2, keepdims=True)
    xxh, xxl = _split(xx)
    yyh, yyl = _split(yy)
    ones = jnp.ones((8, _N, 1), jnp.float32)
    zero = jnp.zeros((8, _N, 1), jnp.float32)
    pa8 = jnp.concatenate(
        [p8, xxh, xxl, ones, ones, zero], axis=2)
    ga8 = jnp.concatenate(
        [-2.0 * g8, ones, ones, yyh, yyl, zero], axis=2)
    ga8 = ga8.transpose(0, 2, 1)

    s1, s2 = pl.pallas_call(
        _chamfer_body,
        grid=(8,),
        in_specs=[
            pl.BlockSpec((1, _N, _KA), lambda b: (b, 0, 0)),
            pl.BlockSpec((1, _KA, _N), lambda b: (b, 0, 0)),
        ],
        out_specs=[
            pl.BlockSpec((1, 1, 128), lambda b: (b, 0, 0)),
            pl.BlockSpec((1, 1, 128), lambda b: (b, 0, 0)),
        ],
        out_shape=[
            jax.ShapeDtypeStruct((8, 1, 128), jnp.float32),
            jax.ShapeDtypeStruct((8, 1, 128), jnp.float32),
        ],
    )(pa8, ga8)

    total = jnp.sum(s1[:, 0, 0]) + jnp.sum(s2[:, 0, 0])
    return total / jnp.float32(4 * _N)


# speedup vs baseline: 1.7517x; 1.2430x over previous
import functools

import jax
import jax.numpy as jnp
from jax.experimental import pallas as pl
from jax.experimental.pallas import tpu as pltpu

_N = 4096
_TP = 256
_KA = 8


def _norm_split(sq):
    hi = jnp.floor(sq * (1.0 / 256.0)) * 256.0
    return hi, sq - hi


def _vox_t(ct):
    cn = jnp.where(jnp.isnan(ct), jnp.inf, ct)
    mn = jnp.min(cn, axis=1, keepdims=True)
    return ((ct - mn) / 0.1).astype(jnp.int32).astype(jnp.float32)


def _chamfer_body(p_ref, g_ref, o_ref, pa_ref, ga_ref):
    p = p_ref[0]
    g = g_ref[0]
    pv = _vox_t(p)
    gv = _vox_t(g)
    shift = jnp.floor(jnp.maximum(jnp.max(pv, axis=1, keepdims=True),
                                  jnp.max(gv, axis=1, keepdims=True)) * 0.5)
    pv = pv - shift
    gv = gv - shift

    one_row = jnp.ones((1, _N), jnp.float32)
    pa_ref[7:8, :] = jnp.zeros((1, _N), jnp.float32)
    ga_ref[7:8, :] = jnp.zeros((1, _N), jnp.float32)
    total = jnp.float32(0.0)
    for pt, gt in ((p, g), (pv, gv)):
        xxh, xxl = _norm_split(jnp.sum(pt * pt, axis=0, keepdims=True))
        yyh, yyl = _norm_split(jnp.sum(gt * gt, axis=0, keepdims=True))
        pa_ref[0:3, :] = pt
        pa_ref[3:4, :] = xxh
        pa_ref[4:5, :] = xxl
        pa_ref[5:6, :] = one_row
        pa_ref[6:7, :] = one_row
        ga_ref[0:3, :] = -2.0 * gt
        ga_ref[3:4, :] = one_row
        ga_ref[4:5, :] = one_row
        ga_ref[5:6, :] = yyh
        ga_ref[6:7, :] = yyl
        ga = ga_ref[...]

        def step(c, carry):
            cacc, s1 = carry
            pc = pa_ref[:, pl.ds(c * _TP, _TP)]
            d = jax.lax.dot_general(
                pc, ga, (((0,), (0,)), ((), ())),
                preferred_element_type=jnp.float32)
            s1 = s1 + jnp.sum(jnp.min(d, axis=1))
            cacc = jnp.minimum(cacc, jnp.min(d, axis=0, keepdims=True))
            return cacc, s1

        cacc0 = jnp.full((1, _N), jnp.inf, dtype=jnp.float32)
        cacc, s1 = jax.lax.fori_loop(
            0, _N // _TP, step, (cacc0, jnp.float32(0.0)), unroll=16)
        total = total + s1 + jnp.sum(cacc)
    o_ref[0, 0, :] = jnp.full((128,), total, dtype=jnp.float32)


@jax.jit
def kernel(preds, gts):
    p_t = preds.transpose(0, 2, 1)
    g_t = gts.transpose(0, 2, 1)

    sums = pl.pallas_call(
        _chamfer_body,
        grid=(4,),
        in_specs=[
            pl.BlockSpec((1, 3, _N), lambda b: (b, 0, 0)),
            pl.BlockSpec((1, 3, _N), lambda b: (b, 0, 0)),
        ],
        out_specs=pl.BlockSpec((1, 1, 128), lambda b: (b, 0, 0)),
        out_shape=jax.ShapeDtypeStruct((4, 1, 128), jnp.float32),
        scratch_shapes=[
            pltpu.VMEM((_KA, _N), jnp.float32),
            pltpu.VMEM((_KA, _N), jnp.float32),
        ],
    )(p_t, g_t)

    return jnp.sum(sums[:, 0, 0]) / jnp.float32(4 * _N)
